# baseline (speedup 1.0000x reference)
import jax
import jax.numpy as jnp
from jax.experimental import pallas as pl
from jax.experimental.pallas import tpu as pltpu

B = 64
V = 100000
D = 128
MAX_SEQ = 2048
CTX = 1
MAX_GEN = MAX_SEQ - CTX
PAD = 0
EOS = 2

NQ = 4
SUB = 2048
VC = NQ * SUB
NCHUNK = (V + VC - 1) // VC
VPADDED = NCHUNK * VC
NEG = -1e30


def _body(s_ref, cur_vec_ref, emb_ref, *rest):
    w_refs = rest[:NQ]
    b_ref, gen_ref, tok_out, buf_out, step_out, h_ref, bv_ref, bi_ref, sem = rest[NQ:]
    i = pl.program_id(0)

    @pl.when(i == 0)
    def _init_and_gather():
        bv_ref[:] = jnp.full((B, 1), NEG, dtype=jnp.float32)
        bi_ref[:] = jnp.zeros((B, 1), dtype=jnp.int32)

        def _start(r, c):
            idx = s_ref[r]
            pltpu.make_async_copy(
                emb_ref.at[pl.ds(idx, 1), :], h_ref.at[pl.ds(r, 1), :], sem
            ).start()
            return c

        jax.lax.fori_loop(0, B, _start, 0)

        def _wait(r, c):
            idx = s_ref[r]
            pltpu.make_async_copy(
                emb_ref.at[pl.ds(idx, 1), :], h_ref.at[pl.ds(r, 1), :], sem
            ).wait()
            return c

        jax.lax.fori_loop(0, B, _wait, 0)

    h = h_ref[:]
    for q in range(NQ):
        logits = jnp.dot(h, w_refs[q][:], preferred_element_type=jnp.float32)
        logits = logits + b_ref[0, q * SUB:(q + 1) * SUB][None, :]
        base = i * VC + q * SUB
        col_ids = base + jax.lax.broadcasted_iota(jnp.int32, (1, SUB), 1)
        if (V - 1) // SUB in range(q, NCHUNK * NQ, NQ):
            logits = jnp.where(col_ids < V, logits, NEG)
        cmax = jnp.max(logits, axis=1, keepdims=True)
        carg = jnp.min(jnp.where(logits == cmax, col_ids, V), axis=1,
                       keepdims=True)
        upd = cmax > bv_ref[:]
        bi_ref[:] = jnp.where(upd, carg.astype(jnp.int32), bi_ref[:])
        bv_ref[:] = jnp.where(upd, cmax, bv_ref[:])

    @pl.when(i == NCHUNK - 1)
    def _finish():
        cur = cur_vec_ref[:]
        tok = jnp.where(cur == EOS, EOS, bi_ref[:])
        tok_out[:] = tok
        col = s_ref[B]
        begin_new = s_ref[B + 1]
        keep = 1.0 - begin_new.astype(jnp.float32)
        base_buf = gen_ref[:] * keep + (1.0 - keep) * jnp.float32(PAD)
        cids = jax.lax.broadcasted_iota(jnp.int32, (B, MAX_GEN), 1)
        add = jnp.where(cids == col, tok.astype(jnp.float32) - jnp.float32(PAD), 0.0)
        buf_out[:] = base_buf + add
        step_out[0] = col.astype(jnp.float32) + 1.0


def kernel(decoder_input_ids, emb, W_out, b_out, generated_tokens, generation_step):
    stepf = generation_step[0]
    stepc = jnp.where(stepf < MAX_GEN, stepf, 0.0)
    begin_new = (stepc == 0.0).astype(jnp.int32)
    col = stepc.astype(jnp.int32)
    prev_col = jnp.maximum(col - 1, 0)
    prev = jax.lax.dynamic_slice(generated_tokens, (0, prev_col), (B, 1))
    cur = jnp.where(begin_new == 1, decoder_input_ids[:, 0],
                    prev[:, 0].astype(jnp.int32))
    scalars = jnp.concatenate([cur, col[None], begin_new[None]])
    cur_vec = cur[:, None]
    b2 = jnp.pad(b_out.reshape(1, V), ((0, 0), (0, VPADDED - V)),
                 constant_values=NEG)

    last_w_block = (V - 1) // SUB

    def _w_spec(q):
        return pl.BlockSpec(
            (D, SUB),
            lambda i, s, q=q: (0, jnp.minimum(NQ * i + q, last_w_block)),
        )

    grid_spec = pltpu.PrefetchScalarGridSpec(
        num_scalar_prefetch=1,
        grid=(NCHUNK,),
        in_specs=[
            pl.BlockSpec((B, 1), lambda i, s: (0, 0)),
            pl.BlockSpec(memory_space=pltpu.HBM),
        ] + [_w_spec(q) for q in range(NQ)] + [
            pl.BlockSpec((1, VC), lambda i, s: (0, i)),
            pl.BlockSpec((B, MAX_GEN), lambda i, s: (0, 0)),
        ],
        out_specs=[
            pl.BlockSpec((B, 1), lambda i, s: (0, 0)),
            pl.BlockSpec((B, MAX_GEN), lambda i, s: (0, 0)),
            pl.BlockSpec(memory_space=pltpu.SMEM),
        ],
        scratch_shapes=[
            pltpu.VMEM((B, D), jnp.float32),
            pltpu.VMEM((B, 1), jnp.float32),
            pltpu.VMEM((B, 1), jnp.int32),
            pltpu.SemaphoreType.DMA,
        ],
    )

    tokens, new_buffer, new_step = pl.pallas_call(
        _body,
        grid_spec=grid_spec,
        out_shape=[
            jax.ShapeDtypeStruct((B, 1), jnp.int32),
            jax.ShapeDtypeStruct((B, MAX_GEN), jnp.float32),
            jax.ShapeDtypeStruct((1,), jnp.float32),
        ],
        compiler_params=pltpu.CompilerParams(
            dimension_semantics=("arbitrary",),
        ),
    )(scalars, cur_vec, emb, *([W_out] * NQ), b2, generated_tokens)
    return tokens, new_buffer, new_step


# speedup vs baseline: 1.0006x; 1.0006x over previous
import jax
import jax.numpy as jnp
from jax.experimental import pallas as pl
from jax.experimental.pallas import tpu as pltpu

B = 64
V = 100000
D = 128
MAX_SEQ = 2048
CTX = 1
MAX_GEN = MAX_SEQ - CTX
PAD = 0
EOS = 2

NQ = 8
SUB = 2048
VC = NQ * SUB
NCHUNK = (V + VC - 1) // VC
VPADDED = NCHUNK * VC
NEG = -1e30


def _body(s_ref, cur_vec_ref, emb_ref, *rest):
    w_refs = rest[:NQ]
    b_ref, gen_ref, tok_out, buf_out, step_out, h_ref, bv_ref, bi_ref, sem = rest[NQ:]
    i = pl.program_id(0)

    @pl.when(i == 0)
    def _init_and_gather():
        bv_ref[:] = jnp.full((B, 1), NEG, dtype=jnp.float32)
        bi_ref[:] = jnp.zeros((B, 1), dtype=jnp.int32)

        def _start(r, c):
            idx = s_ref[r]
            pltpu.make_async_copy(
                emb_ref.at[pl.ds(idx, 1), :], h_ref.at[pl.ds(r, 1), :], sem
            ).start()
            return c

        jax.lax.fori_loop(0, B, _start, 0)

        def _wait(r, c):
            idx = s_ref[r]
            pltpu.make_async_copy(
                emb_ref.at[pl.ds(idx, 1), :], h_ref.at[pl.ds(r, 1), :], sem
            ).wait()
            return c

        jax.lax.fori_loop(0, B, _wait, 0)

    h = h_ref[:]
    for q in range(NQ):
        logits = jnp.dot(h, w_refs[q][:], preferred_element_type=jnp.float32)
        logits = logits + b_ref[0, q * SUB:(q + 1) * SUB][None, :]
        base = i * VC + q * SUB
        col_ids = base + jax.lax.broadcasted_iota(jnp.int32, (1, SUB), 1)
        if (V - 1) // SUB in range(q, NCHUNK * NQ, NQ):
            logits = jnp.where(col_ids < V, logits, NEG)
        cmax = jnp.max(logits, axis=1, keepdims=True)
        carg = jnp.min(jnp.where(logits == cmax, col_ids, V), axis=1,
                       keepdims=True)
        upd = cmax > bv_ref[:]
        bi_ref[:] = jnp.where(upd, carg.astype(jnp.int32), bi_ref[:])
        bv_ref[:] = jnp.where(upd, cmax, bv_ref[:])

    @pl.when(i == NCHUNK - 1)
    def _finish():
        cur = cur_vec_ref[:]
        tok = jnp.where(cur == EOS, EOS, bi_ref[:])
        tok_out[:] = tok
        col = s_ref[B]
        begin_new = s_ref[B + 1]
        keep = 1.0 - begin_new.astype(jnp.float32)
        base_buf = gen_ref[:] * keep + (1.0 - keep) * jnp.float32(PAD)
        cids = jax.lax.broadcasted_iota(jnp.int32, (B, MAX_GEN), 1)
        add = jnp.where(cids == col, tok.astype(jnp.float32) - jnp.float32(PAD), 0.0)
        buf_out[:] = base_buf + add
        step_out[0] = col.astype(jnp.float32) + 1.0


def kernel(decoder_input_ids, emb, W_out, b_out, generated_tokens, generation_step):
    stepf = generation_step[0]
    stepc = jnp.where(stepf < MAX_GEN, stepf, 0.0)
    begin_new = (stepc == 0.0).astype(jnp.int32)
    col = stepc.astype(jnp.int32)
    prev_col = jnp.maximum(col - 1, 0)
    prev = jax.lax.dynamic_slice(generated_tokens, (0, prev_col), (B, 1))
    cur = jnp.where(begin_new == 1, decoder_input_ids[:, 0],
                    prev[:, 0].astype(jnp.int32))
    scalars = jnp.concatenate([cur, col[None], begin_new[None]])
    cur_vec = cur[:, None]
    b2 = jnp.pad(b_out.reshape(1, V), ((0, 0), (0, VPADDED - V)),
                 constant_values=NEG)

    last_w_block = (V - 1) // SUB

    def _w_spec(q):
        return pl.BlockSpec(
            (D, SUB),
            lambda i, s, q=q: (0, jnp.minimum(NQ * i + q, last_w_block)),
        )

    grid_spec = pltpu.PrefetchScalarGridSpec(
        num_scalar_prefetch=1,
        grid=(NCHUNK,),
        in_specs=[
            pl.BlockSpec((B, 1), lambda i, s: (0, 0)),
            pl.BlockSpec(memory_space=pltpu.HBM),
        ] + [_w_spec(q) for q in range(NQ)] + [
            pl.BlockSpec((1, VC), lambda i, s: (0, i)),
            pl.BlockSpec((B, MAX_GEN), lambda i, s: (0, 0)),
        ],
        out_specs=[
            pl.BlockSpec((B, 1), lambda i, s: (0, 0)),
            pl.BlockSpec((B, MAX_GEN), lambda i, s: (0, 0)),
            pl.BlockSpec(memory_space=pltpu.SMEM),
        ],
        scratch_shapes=[
            pltpu.VMEM((B, D), jnp.float32),
            pltpu.VMEM((B, 1), jnp.float32),
            pltpu.VMEM((B, 1), jnp.int32),
            pltpu.SemaphoreType.DMA,
        ],
    )

    tokens, new_buffer, new_step = pl.pallas_call(
        _body,
        grid_spec=grid_spec,
        out_shape=[
            jax.ShapeDtypeStruct((B, 1), jnp.int32),
            jax.ShapeDtypeStruct((B, MAX_GEN), jnp.float32),
            jax.ShapeDtypeStruct((1,), jnp.float32),
        ],
        compiler_params=pltpu.CompilerParams(
            dimension_semantics=("arbitrary",),
        ),
    )(scalars, cur_vec, emb, *([W_out] * NQ), b2, generated_tokens)
    return tokens, new_buffer, new_step


# speedup vs baseline: 1.0105x; 1.0099x over previous
import jax
import jax.numpy as jnp
from jax.experimental import pallas as pl
from jax.experimental.pallas import tpu as pltpu

B = 64
V = 100000
D = 128
MAX_SEQ = 2048
CTX = 1
MAX_GEN = MAX_SEQ - CTX
PAD = 0
EOS = 2

NQ = 1
SUB = 8192
VC = NQ * SUB
NCHUNK = (V + VC - 1) // VC
VPADDED = NCHUNK * VC
NEG = -1e30


def _body(s_ref, cur_vec_ref, emb_ref, *rest):
    w_refs = rest[:NQ]
    b_ref, gen_ref, tok_out, buf_out, step_out, h_ref, bv_ref, bi_ref, sem = rest[NQ:]
    i = pl.program_id(0)

    @pl.when(i == 0)
    def _init_and_gather():
        bv_ref[:] = jnp.full((B, 1), NEG, dtype=jnp.float32)
        bi_ref[:] = jnp.zeros((B, 1), dtype=jnp.int32)

        def _start(r, c):
            idx = s_ref[r]
            pltpu.make_async_copy(
                emb_ref.at[pl.ds(idx, 1), :], h_ref.at[pl.ds(r, 1), :], sem
            ).start()
            return c

        jax.lax.fori_loop(0, B, _start, 0)

        def _wait(r, c):
            idx = s_ref[r]
            pltpu.make_async_copy(
                emb_ref.at[pl.ds(idx, 1), :], h_ref.at[pl.ds(r, 1), :], sem
            ).wait()
            return c

        jax.lax.fori_loop(0, B, _wait, 0)

    h = h_ref[:]
    for q in range(NQ):
        logits = jnp.dot(h, w_refs[q][:], preferred_element_type=jnp.float32)
        logits = logits + b_ref[0, q * SUB:(q + 1) * SUB][None, :]
        base = i * VC + q * SUB
        col_ids = base + jax.lax.broadcasted_iota(jnp.int32, (1, SUB), 1)
        if (V - 1) // SUB in range(q, NCHUNK * NQ, NQ):
            logits = jnp.where(col_ids < V, logits, NEG)
        cmax = jnp.max(logits, axis=1, keepdims=True)
        carg = jnp.min(jnp.where(logits == cmax, col_ids, V), axis=1,
                       keepdims=True)
        upd = cmax > bv_ref[:]
        bi_ref[:] = jnp.where(upd, carg.astype(jnp.int32), bi_ref[:])
        bv_ref[:] = jnp.where(upd, cmax, bv_ref[:])

    @pl.when(i == NCHUNK - 1)
    def _finish():
        cur = cur_vec_ref[:]
        tok = jnp.where(cur == EOS, EOS, bi_ref[:])
        tok_out[:] = tok
        col = s_ref[B]
        begin_new = s_ref[B + 1]
        keep = 1.0 - begin_new.astype(jnp.float32)
        base_buf = gen_ref[:] * keep + (1.0 - keep) * jnp.float32(PAD)
        cids = jax.lax.broadcasted_iota(jnp.int32, (B, MAX_GEN), 1)
        add = jnp.where(cids == col, tok.astype(jnp.float32) - jnp.float32(PAD), 0.0)
        buf_out[:] = base_buf + add
        step_out[0] = col.astype(jnp.float32) + 1.0


def kernel(decoder_input_ids, emb, W_out, b_out, generated_tokens, generation_step):
    stepf = generation_step[0]
    stepc = jnp.where(stepf < MAX_GEN, stepf, 0.0)
    begin_new = (stepc == 0.0).astype(jnp.int32)
    col = stepc.astype(jnp.int32)
    prev_col = jnp.maximum(col - 1, 0)
    prev = jax.lax.dynamic_slice(generated_tokens, (0, prev_col), (B, 1))
    cur = jnp.where(begin_new == 1, decoder_input_ids[:, 0],
                    prev[:, 0].astype(jnp.int32))
    scalars = jnp.concatenate([cur, col[None], begin_new[None]])
    cur_vec = cur[:, None]
    b2 = jnp.pad(b_out.reshape(1, V), ((0, 0), (0, VPADDED - V)),
                 constant_values=NEG)

    last_w_block = (V - 1) // SUB

    def _w_spec(q):
        return pl.BlockSpec(
            (D, SUB),
            lambda i, s, q=q: (0, jnp.minimum(NQ * i + q, last_w_block)),
        )

    grid_spec = pltpu.PrefetchScalarGridSpec(
        num_scalar_prefetch=1,
        grid=(NCHUNK,),
        in_specs=[
            pl.BlockSpec((B, 1), lambda i, s: (0, 0)),
            pl.BlockSpec(memory_space=pltpu.HBM),
        ] + [_w_spec(q) for q in range(NQ)] + [
            pl.BlockSpec((1, VC), lambda i, s: (0, i)),
            pl.BlockSpec((B, MAX_GEN), lambda i, s: (0, 0)),
        ],
        out_specs=[
            pl.BlockSpec((B, 1), lambda i, s: (0, 0)),
            pl.BlockSpec((B, MAX_GEN), lambda i, s: (0, 0)),
            pl.BlockSpec(memory_space=pltpu.SMEM),
        ],
        scratch_shapes=[
            pltpu.VMEM((B, D), jnp.float32),
            pltpu.VMEM((B, 1), jnp.float32),
            pltpu.VMEM((B, 1), jnp.int32),
            pltpu.SemaphoreType.DMA,
        ],
    )

    tokens, new_buffer, new_step = pl.pallas_call(
        _body,
        grid_spec=grid_spec,
        out_shape=[
            jax.ShapeDtypeStruct((B, 1), jnp.int32),
            jax.ShapeDtypeStruct((B, MAX_GEN), jnp.float32),
            jax.ShapeDtypeStruct((1,), jnp.float32),
        ],
        compiler_params=pltpu.CompilerParams(
            dimension_semantics=("arbitrary",),
        ),
    )(scalars, cur_vec, emb, *([W_out] * NQ), b2, generated_tokens)
    return tokens, new_buffer, new_step


# speedup vs baseline: 1.0239x; 1.0133x over previous
import jax
import jax.numpy as jnp
from jax.experimental import pallas as pl
from jax.experimental.pallas import tpu as pltpu

B = 64
V = 100000
D = 128
MAX_SEQ = 2048
CTX = 1
MAX_GEN = MAX_SEQ - CTX
PAD = 0
EOS = 2

NQ = 1
SUB = 16384
VC = NQ * SUB
NCHUNK = (V + VC - 1) // VC
VPADDED = NCHUNK * VC
NEG = -1e30


def _body(s_ref, cur_vec_ref, emb_ref, *rest):
    w_refs = rest[:NQ]
    b_ref, gen_ref, tok_out, buf_out, step_out, h_ref, bv_ref, bi_ref, sem = rest[NQ:]
    i = pl.program_id(0)

    @pl.when(i == 0)
    def _init_and_gather():
        bv_ref[:] = jnp.full((B, 1), NEG, dtype=jnp.float32)
        bi_ref[:] = jnp.zeros((B, 1), dtype=jnp.int32)

        def _start(r, c):
            idx = s_ref[r]
            pltpu.make_async_copy(
                emb_ref.at[pl.ds(idx, 1), :], h_ref.at[pl.ds(r, 1), :], sem
            ).start()
            return c

        jax.lax.fori_loop(0, B, _start, 0)

        def _wait(r, c):
            idx = s_ref[r]
            pltpu.make_async_copy(
                emb_ref.at[pl.ds(idx, 1), :], h_ref.at[pl.ds(r, 1), :], sem
            ).wait()
            return c

        jax.lax.fori_loop(0, B, _wait, 0)

    h = h_ref[:]
    for q in range(NQ):
        logits = jnp.dot(h, w_refs[q][:], preferred_element_type=jnp.float32)
        logits = logits + b_ref[0, q * SUB:(q + 1) * SUB][None, :]
        base = i * VC + q * SUB
        col_ids = base + jax.lax.broadcasted_iota(jnp.int32, (1, SUB), 1)
        if (V - 1) // SUB in range(q, NCHUNK * NQ, NQ):
            logits = jnp.where(col_ids < V, logits, NEG)
        cmax = jnp.max(logits, axis=1, keepdims=True)
        carg = jnp.min(jnp.where(logits == cmax, col_ids, V), axis=1,
                       keepdims=True)
        upd = cmax > bv_ref[:]
        bi_ref[:] = jnp.where(upd, carg.astype(jnp.int32), bi_ref[:])
        bv_ref[:] = jnp.where(upd, cmax, bv_ref[:])

    @pl.when(i == NCHUNK - 1)
    def _finish():
        cur = cur_vec_ref[:]
        tok = jnp.where(cur == EOS, EOS, bi_ref[:])
        tok_out[:] = tok
        col = s_ref[B]
        begin_new = s_ref[B + 1]
        keep = 1.0 - begin_new.astype(jnp.float32)
        base_buf = gen_ref[:] * keep + (1.0 - keep) * jnp.float32(PAD)
        cids = jax.lax.broadcasted_iota(jnp.int32, (B, MAX_GEN), 1)
        add = jnp.where(cids == col, tok.astype(jnp.float32) - jnp.float32(PAD), 0.0)
        buf_out[:] = base_buf + add
        step_out[0] = col.astype(jnp.float32) + 1.0


def kernel(decoder_input_ids, emb, W_out, b_out, generated_tokens, generation_step):
    stepf = generation_step[0]
    stepc = jnp.where(stepf < MAX_GEN, stepf, 0.0)
    begin_new = (stepc == 0.0).astype(jnp.int32)
    col = stepc.astype(jnp.int32)
    prev_col = jnp.maximum(col - 1, 0)
    prev = jax.lax.dynamic_slice(generated_tokens, (0, prev_col), (B, 1))
    cur = jnp.where(begin_new == 1, decoder_input_ids[:, 0],
                    prev[:, 0].astype(jnp.int32))
    scalars = jnp.concatenate([cur, col[None], begin_new[None]])
    cur_vec = cur[:, None]
    b2 = jnp.pad(b_out.reshape(1, V), ((0, 0), (0, VPADDED - V)),
                 constant_values=NEG)

    last_w_block = (V - 1) // SUB

    def _w_spec(q):
        return pl.BlockSpec(
            (D, SUB),
            lambda i, s, q=q: (0, jnp.minimum(NQ * i + q, last_w_block)),
        )

    grid_spec = pltpu.PrefetchScalarGridSpec(
        num_scalar_prefetch=1,
        grid=(NCHUNK,),
        in_specs=[
            pl.BlockSpec((B, 1), lambda i, s: (0, 0)),
            pl.BlockSpec(memory_space=pltpu.HBM),
        ] + [_w_spec(q) for q in range(NQ)] + [
            pl.BlockSpec((1, VC), lambda i, s: (0, i)),
            pl.BlockSpec((B, MAX_GEN), lambda i, s: (0, 0)),
        ],
        out_specs=[
            pl.BlockSpec((B, 1), lambda i, s: (0, 0)),
            pl.BlockSpec((B, MAX_GEN), lambda i, s: (0, 0)),
            pl.BlockSpec(memory_space=pltpu.SMEM),
        ],
        scratch_shapes=[
            pltpu.VMEM((B, D), jnp.float32),
            pltpu.VMEM((B, 1), jnp.float32),
            pltpu.VMEM((B, 1), jnp.int32),
            pltpu.SemaphoreType.DMA,
        ],
    )

    tokens, new_buffer, new_step = pl.pallas_call(
        _body,
        grid_spec=grid_spec,
        out_shape=[
            jax.ShapeDtypeStruct((B, 1), jnp.int32),
            jax.ShapeDtypeStruct((B, MAX_GEN), jnp.float32),
            jax.ShapeDtypeStruct((1,), jnp.float32),
        ],
        compiler_params=pltpu.CompilerParams(
            dimension_semantics=("arbitrary",),
        ),
    )(scalars, cur_vec, emb, *([W_out] * NQ), b2, generated_tokens)
    return tokens, new_buffer, new_step


# speedup vs baseline: 1.0264x; 1.0025x over previous
import jax
import jax.numpy as jnp
from jax.experimental import pallas as pl
from jax.experimental.pallas import tpu as pltpu

B = 64
V = 100000
D = 128
MAX_SEQ = 2048
CTX = 1
MAX_GEN = MAX_SEQ - CTX
PAD = 0
EOS = 2

CW = 8192
NBUF = 4
VAL = (V // 128) * 128
VREM = V - VAL
NCH = (VAL + CW - 1) // CW
TAIL = VAL - (NCH - 1) * CW
VPADDED = NCH * CW
NEG = -1e30


def _body(s_ref, cur_vec_ref, emb_ref, w_ref, wt_ref, b_ref, bt_ref, gen_ref,
          tok_out, buf_out, step_out, *scratch):
    h_ref = scratch[0]
    wbufs = scratch[1:1 + NBUF]
    sems = scratch[1 + NBUF:1 + 2 * NBUF]
    gsem = scratch[1 + 2 * NBUF]

    def _gstart(r, c):
        idx = s_ref[r]
        pltpu.make_async_copy(
            emb_ref.at[pl.ds(idx, 1), :], h_ref.at[pl.ds(r, 1), :], gsem
        ).start()
        return c

    jax.lax.fori_loop(0, B, _gstart, 0)

    def _wcopy(c):
        slot = c % NBUF
        if c == NCH - 1:
            return pltpu.make_async_copy(
                w_ref.at[:, pl.ds(c * CW, TAIL)],
                wbufs[slot].at[:, pl.ds(0, TAIL)],
                sems[slot],
            )
        return pltpu.make_async_copy(
            w_ref.at[:, pl.ds(c * CW, CW)], wbufs[slot], sems[slot]
        )

    for c in range(min(NBUF, NCH)):
        _wcopy(c).start()

    def _gwait(r, c):
        idx = s_ref[r]
        pltpu.make_async_copy(
            emb_ref.at[pl.ds(idx, 1), :], h_ref.at[pl.ds(r, 1), :], gsem
        ).wait()
        return c

    jax.lax.fori_loop(0, B, _gwait, 0)
    h = h_ref[:]

    tl = jnp.dot(h, wt_ref[:], preferred_element_type=jnp.float32)
    tl = tl + bt_ref[0, :][None, :]
    tcol = VAL + jax.lax.broadcasted_iota(jnp.int32, (1, VREM), 1)
    bv = jnp.max(tl, axis=1, keepdims=True)
    bi = jnp.min(jnp.where(tl == bv, tcol, V), axis=1, keepdims=True)
    for c in range(NCH):
        slot = c % NBUF
        _wcopy(c).wait()
        logits = jnp.dot(h, wbufs[slot][:], preferred_element_type=jnp.float32)
        logits = logits + b_ref[0, c * CW:(c + 1) * CW][None, :]
        nxt = c + NBUF
        if nxt < NCH:
            _wcopy(nxt).start()
        col_ids = c * CW + jax.lax.broadcasted_iota(jnp.int32, (1, CW), 1)
        cmax = jnp.max(logits, axis=1, keepdims=True)
        carg = jnp.min(jnp.where(logits == cmax, col_ids, V), axis=1,
                       keepdims=True)
        carg = carg.astype(jnp.int32)
        upd = (cmax > bv) | ((cmax == bv) & (carg < bi))
        bi = jnp.where(upd, carg, bi)
        bv = jnp.where(upd, cmax, bv)

    cur = cur_vec_ref[:]
    tok = jnp.where(cur == EOS, EOS, bi)
    tok_out[:] = tok
    col = s_ref[B]
    begin_new = s_ref[B + 1]
    keep = 1.0 - begin_new.astype(jnp.float32)
    base_buf = gen_ref[:] * keep + (1.0 - keep) * jnp.float32(PAD)
    cids = jax.lax.broadcasted_iota(jnp.int32, (B, MAX_GEN), 1)
    add = jnp.where(cids == col, tok.astype(jnp.float32) - jnp.float32(PAD), 0.0)
    buf_out[:] = base_buf + add
    step_out[0] = col.astype(jnp.float32) + 1.0


def kernel(decoder_input_ids, emb, W_out, b_out, generated_tokens, generation_step):
    stepf = generation_step[0]
    stepc = jnp.where(stepf < MAX_GEN, stepf, 0.0)
    begin_new = (stepc == 0.0).astype(jnp.int32)
    col = stepc.astype(jnp.int32)
    prev_col = jnp.maximum(col - 1, 0)
    prev = jax.lax.dynamic_slice(generated_tokens, (0, prev_col), (B, 1))
    cur = jnp.where(begin_new == 1, decoder_input_ids[:, 0],
                    prev[:, 0].astype(jnp.int32))
    scalars = jnp.concatenate([cur, col[None], begin_new[None]])
    cur_vec = cur[:, None]
    b2 = jnp.pad(b_out[:VAL].reshape(1, VAL), ((0, 0), (0, VPADDED - VAL)),
                 constant_values=NEG)
    btail = b_out[VAL:].reshape(1, VREM)
    wtail = jax.lax.slice(W_out, (0, VAL), (D, V))

    grid_spec = pltpu.PrefetchScalarGridSpec(
        num_scalar_prefetch=1,
        grid=(1,),
        in_specs=[
            pl.BlockSpec((B, 1), lambda i, s: (0, 0)),
            pl.BlockSpec(memory_space=pltpu.HBM),
            pl.BlockSpec(memory_space=pltpu.HBM),
            pl.BlockSpec((D, VREM), lambda i, s: (0, 0)),
            pl.BlockSpec((1, VPADDED), lambda i, s: (0, 0)),
            pl.BlockSpec((1, VREM), lambda i, s: (0, 0)),
            pl.BlockSpec((B, MAX_GEN), lambda i, s: (0, 0)),
        ],
        out_specs=[
            pl.BlockSpec((B, 1), lambda i, s: (0, 0)),
            pl.BlockSpec((B, MAX_GEN), lambda i, s: (0, 0)),
            pl.BlockSpec(memory_space=pltpu.SMEM),
        ],
        scratch_shapes=[pltpu.VMEM((B, D), jnp.float32)]
        + [pltpu.VMEM((D, CW), jnp.float32) for _ in range(NBUF)]
        + [pltpu.SemaphoreType.DMA for _ in range(NBUF)]
        + [pltpu.SemaphoreType.DMA],
    )

    tokens, new_buffer, new_step = pl.pallas_call(
        _body,
        grid_spec=grid_spec,
        out_shape=[
            jax.ShapeDtypeStruct((B, 1), jnp.int32),
            jax.ShapeDtypeStruct((B, MAX_GEN), jnp.float32),
            jax.ShapeDtypeStruct((1,), jnp.float32),
        ],
        compiler_params=pltpu.CompilerParams(
            dimension_semantics=("arbitrary",),
        ),
    )(scalars, cur_vec, emb, W_out, wtail, b2, btail, generated_tokens)
    return tokens, new_buffer, new_step
